# Initial kernel scaffold; baseline (speedup 1.0000x reference)
import jax
import jax.numpy as jnp
from jax.experimental import pallas as pl


def kernel(x, token_table, pos_table):
    raise NotImplementedError("write your pallas kernel here")


# speedup vs baseline: 3.6276x; 3.6276x over previous
import functools

import jax
import jax.numpy as jnp
from jax import lax
from jax.experimental import pallas as pl
from jax.experimental.pallas import tpu as pltpu
from jax.experimental.pallas import tpu_sc as plsc

VOCAB = 100000
MAX_LEN = 200
EMBED_DIM = 64
BATCH = 4096

_INFO = plsc.get_sparse_core_info()
NUM_CORES = _INFO.num_cores
NUM_SUBCORES = _INFO.num_subcores
NUM_WORKERS = NUM_CORES * NUM_SUBCORES

TOTAL = BATCH * MAX_LEN
PER_WORKER = TOTAL // NUM_WORKERS
SEQS_PER_WORKER = PER_WORKER // MAX_LEN
CHUNK_SEQS = 4
CHUNK_ROWS = CHUNK_SEQS * MAX_LEN
NUM_CHUNKS = SEQS_PER_WORKER // CHUNK_SEQS
LANES = 16
VECS_PER_ROW = EMBED_DIM // LANES


def _body(x_hbm, tok_hbm, pos_hbm, out_hbm, idx_v, pos_v, rows_v, sem):
    wid = lax.axis_index("s") * NUM_CORES + lax.axis_index("c")
    base = wid * PER_WORKER

    pltpu.sync_copy(x_hbm.at[pl.ds(base, PER_WORKER)], idx_v)
    pltpu.sync_copy(pos_hbm, pos_v)

    def chunk_body(c, _):
        row0 = c * CHUNK_ROWS
        pltpu.async_copy(
            tok_hbm.at[idx_v.at[pl.ds(row0, CHUNK_ROWS)]], rows_v, sem
        ).wait()

        def add_body(j, _):
            for s in range(CHUNK_SEQS):
                r = s * MAX_LEN + j
                for k in range(VECS_PER_ROW):
                    sl = pl.ds(k * LANES, LANES)
                    rows_v[r, sl] = rows_v[r, sl] + pos_v[j, sl]
            return _

        lax.fori_loop(0, MAX_LEN, add_body, None)

        pltpu.sync_copy(rows_v, out_hbm.at[pl.ds(base + row0, CHUNK_ROWS)])
        return _

    lax.fori_loop(0, NUM_CHUNKS, chunk_body, None)


def kernel(x, token_table, pos_table):
    x_flat = x.reshape(-1).astype(jnp.int32)

    mesh = plsc.VectorSubcoreMesh(core_axis_name="c", subcore_axis_name="s")
    run = functools.partial(
        pl.kernel,
        out_type=jax.ShapeDtypeStruct((TOTAL, EMBED_DIM), jnp.float32),
        mesh=mesh,
        scratch_types=[
            pltpu.VMEM((PER_WORKER,), jnp.int32),
            pltpu.VMEM((MAX_LEN, EMBED_DIM), jnp.float32),
            pltpu.VMEM((CHUNK_ROWS, EMBED_DIM), jnp.float32),
            pltpu.SemaphoreType.DMA,
        ],
        compiler_params=pltpu.CompilerParams(use_tc_tiling_on_sc=False),
    )(_body)

    out = run(x_flat, token_table, pos_table)
    return out.reshape(BATCH, MAX_LEN, EMBED_DIM)


# speedup vs baseline: 4.1237x; 1.1368x over previous
import functools

import jax
import jax.numpy as jnp
from jax import lax
from jax.experimental import pallas as pl
from jax.experimental.pallas import tpu as pltpu
from jax.experimental.pallas import tpu_sc as plsc

VOCAB = 100000
MAX_LEN = 200
EMBED_DIM = 64
BATCH = 4096

_INFO = plsc.get_sparse_core_info()
NUM_CORES = _INFO.num_cores
NUM_SUBCORES = _INFO.num_subcores
NUM_WORKERS = NUM_CORES * NUM_SUBCORES

TOTAL = BATCH * MAX_LEN
PER_WORKER = TOTAL // NUM_WORKERS
SEQS_PER_WORKER = PER_WORKER // MAX_LEN
CHUNK_SEQS = 2
CHUNK_ROWS = CHUNK_SEQS * MAX_LEN
NUM_CHUNKS = SEQS_PER_WORKER // CHUNK_SEQS
LANES = 16
VECS_PER_ROW = EMBED_DIM // LANES


def _body(x_hbm, tok_hbm, pos_hbm, out_hbm,
          idx_v, pos_v, rows0, rows1, sem0, sem1):
    wid = lax.axis_index("s") * NUM_CORES + lax.axis_index("c")
    base = wid * PER_WORKER

    pltpu.sync_copy(x_hbm.at[pl.ds(base, PER_WORKER)], idx_v)
    pltpu.sync_copy(pos_hbm, pos_v)

    bufs = (rows0, rows1)
    sems = (sem0, sem1)

    def start_gather(c, b):
        pltpu.async_copy(
            tok_hbm.at[idx_v.at[pl.ds(c * CHUNK_ROWS, CHUNK_ROWS)]],
            bufs[b], sems[b],
        )

    def drain_gather(b):
        pltpu.make_async_copy(
            tok_hbm.at[idx_v.at[pl.ds(0, CHUNK_ROWS)]], bufs[b], sems[b]
        ).wait()

    start_gather(0, 0)
    start_gather(1, 1)

    def process(c, b):
        drain_gather(b)

        def add_body(j, carry):
            for s in range(CHUNK_SEQS):
                r = s * MAX_LEN + j
                for k in range(VECS_PER_ROW):
                    sl = pl.ds(k * LANES, LANES)
                    bufs[b][r, sl] = bufs[b][r, sl] + pos_v[j, sl]
            return carry

        lax.fori_loop(0, MAX_LEN, add_body, None)

        pltpu.sync_copy(bufs[b], out_hbm.at[pl.ds(base + c * CHUNK_ROWS,
                                                  CHUNK_ROWS)])

        @pl.when(c + 2 < NUM_CHUNKS)
        def _():
            start_gather(c + 2, b)

    def pair_body(g, carry):
        for b in range(2):
            process(2 * g + b, b)
        return carry

    lax.fori_loop(0, NUM_CHUNKS // 2, pair_body, None)


def kernel(x, token_table, pos_table):
    x_flat = x.reshape(-1).astype(jnp.int32)

    mesh = plsc.VectorSubcoreMesh(core_axis_name="c", subcore_axis_name="s")
    run = functools.partial(
        pl.kernel,
        out_type=jax.ShapeDtypeStruct((TOTAL, EMBED_DIM), jnp.float32),
        mesh=mesh,
        scratch_types=[
            pltpu.VMEM((PER_WORKER,), jnp.int32),
            pltpu.VMEM((MAX_LEN, EMBED_DIM), jnp.float32),
            pltpu.VMEM((CHUNK_ROWS, EMBED_DIM), jnp.float32),
            pltpu.VMEM((CHUNK_ROWS, EMBED_DIM), jnp.float32),
            pltpu.SemaphoreType.DMA,
            pltpu.SemaphoreType.DMA,
        ],
        compiler_params=pltpu.CompilerParams(use_tc_tiling_on_sc=False),
    )(_body)

    out = run(x_flat, token_table, pos_table)
    return out.reshape(BATCH, MAX_LEN, EMBED_DIM)
